# baseline (speedup 1.0000x reference)
import functools

import jax
import jax.numpy as jnp
from jax import lax
from jax.experimental import pallas as pl
from jax.experimental.pallas import tpu as pltpu
from jax.experimental.pallas import tpu_sc as plsc

N = 10000
E = 320000
F_IN = 128
HID = 64
NCLS = 40

NC = 2
NS = 16
NW = NC * NS
N_PAD = 10240
CH = 128
PER_W = 10240
E_PAD = PER_W * NW
NB = PER_W // CH
ROWS_PER_TILE = N_PAD // NS


def _wid():
    return lax.axis_index("s") * NC + lax.axis_index("c")


def _sc_degree(dst_hbm, out_hbm, idx_v, deg_v, sem):
    cid = lax.axis_index("c")
    sid = lax.axis_index("s")
    wid = sid * NC + cid

    def zero_body(i, _):
        deg_v[pl.ds(i * 16, 16)] = jnp.zeros((16,), jnp.float32)
        return 0
    lax.fori_loop(0, N_PAD // 16, zero_body, 0)

    pltpu.sync_copy(dst_hbm.at[pl.ds(wid * PER_W, PER_W)], idx_v)

    ones = jnp.ones((16,), jnp.float32)

    def count_body(i, _):
        idx = idx_v[pl.ds(i * 16, 16)]
        plsc.addupdate_scatter(deg_v, [idx], ones)
        return 0
    lax.fori_loop(0, PER_W // 16, count_body, 0)

    pltpu.sync_copy(deg_v, out_hbm.at[wid])


def _sc_scatter(src_hbm, dst_hbm, y_hbm, out_hbm, idx_s, idx_d,
                rows0, rows1, rows2, rows3, zbuf, acc_sh,
                gsem0, gsem1, gsem2, gsem3,
                ssem0, ssem1, ssem2, ssem3, semi):
    cid = lax.axis_index("c")
    sid = lax.axis_index("s")
    wid = sid * NC + cid

    cps = pltpu.async_copy(src_hbm.at[wid], idx_s, semi)
    cpd = pltpu.async_copy(dst_hbm.at[wid], idx_d, semi)

    def zero_zb(i, _):
        r = i // (HID // 16)
        c = i % (HID // 16)
        zbuf[r, pl.ds(c * 16, 16)] = jnp.zeros((16,), jnp.float32)
        return 0
    lax.fori_loop(0, CH * HID // 16, zero_zb, 0)

    def zero_acc(i, _):
        pltpu.sync_copy(
            zbuf, acc_sh.at[pl.ds(sid * ROWS_PER_TILE + i * CH, CH), :])
        return 0
    lax.fori_loop(0, ROWS_PER_TILE // CH, zero_acc, 0)

    cps.wait()
    cpd.wait()
    plsc.subcore_barrier()

    rows = (rows0, rows1, rows2, rows3)
    gsem = (gsem0, gsem1, gsem2, gsem3)
    ssem = (ssem0, ssem1, ssem2, ssem3)

    def gstart(j, k):
        pltpu.async_copy(y_hbm.at[idx_s.at[j]], rows[k], gsem[k])

    def gwait(k):
        pltpu.make_async_copy(y_hbm.at[idx_s.at[0]], rows[k], gsem[k]).wait()

    def sstart(j, k):
        pltpu.async_copy(rows[k], acc_sh.at[idx_d.at[j]], ssem[k], add=True)

    def swait(k):
        pltpu.make_async_copy(
            rows[k], acc_sh.at[idx_d.at[0]], ssem[k]).wait()

    gstart(0, 0)
    gstart(1, 1)
    for k in range(4):
        gwait(k)
        sstart(k, k)
        if k >= 2:
            swait((k + 2) % 4)
        gstart(k + 2, (k + 2) % 4)

    def grp_body(g, _):
        for k in range(4):
            j = 4 * g + k
            gwait(k)
            sstart(j, k)
            swait((k + 2) % 4)
            gstart(j + 2, (k + 2) % 4)
        return 0
    lax.fori_loop(1, NB // 4 - 1, grp_body, 0)

    for k in range(4):
        j = NB - 4 + k
        gwait(k)
        sstart(j, k)
        if k < 2:
            swait((k + 2) % 4)
            gstart(j + 2, (k + 2) % 4)
    for k in range(4):
        swait(k)

    plsc.subcore_barrier()
    pltpu.sync_copy(
        acc_sh.at[pl.ds(sid * ROWS_PER_TILE, ROWS_PER_TILE), :],
        out_hbm.at[cid, pl.ds(sid * ROWS_PER_TILE, ROWS_PER_TILE), :],
    )


@functools.cache
def _sc_mesh():
    return plsc.VectorSubcoreMesh(
        core_axis_name="c", subcore_axis_name="s",
        num_cores=NC, num_subcores=NS)


@functools.cache
def _deg_kernel():
    return pl.kernel(
        _sc_degree,
        out_type=jax.ShapeDtypeStruct((NW, N_PAD), jnp.float32),
        mesh=_sc_mesh(),
        compiler_params=pltpu.CompilerParams(
            needs_layout_passes=False, use_tc_tiling_on_sc=False),
        scratch_types=[
            pltpu.VMEM((PER_W,), jnp.int32),
            pltpu.VMEM((N_PAD,), jnp.float32),
            pltpu.SemaphoreType.DMA,
        ],
    )


@functools.cache
def _scatter_kernel():
    return pl.kernel(
        _sc_scatter,
        out_type=jax.ShapeDtypeStruct((NC, N_PAD, HID), jnp.float32),
        mesh=_sc_mesh(),
        compiler_params=pltpu.CompilerParams(
            needs_layout_passes=False, use_tc_tiling_on_sc=False),
        scratch_types=(
            [pltpu.VMEM((NB, CH), jnp.int32)] * 2
            + [pltpu.VMEM((CH, HID), jnp.float32)] * 5
            + [pltpu.VMEM_SHARED((N_PAD, HID), jnp.float32)]
            + [pltpu.SemaphoreType.DMA] * 9
        ),
    )


_BLK = 2048
_GRID = N_PAD // _BLK


def _tc1_body(dp_ref, x_ref, w1_ref, y1_ref, dinv_ref):
    deg = jnp.sum(dp_ref[...], axis=0) + 1.0
    dinv = lax.rsqrt(deg)
    dinv_ref[...] = dinv
    xw = jnp.dot(x_ref[...], w1_ref[...], preferred_element_type=jnp.float32)
    y1_ref[...] = xw * dinv[:, None]


def _tc2_body(a_ref, y1_ref, dinv_ref, w2_ref, b1_ref, y2_ref):
    dinv = dinv_ref[...]
    pre = (a_ref[0] + a_ref[1] + y1_ref[...]) * dinv[:, None]
    h = jnp.maximum(pre + b1_ref[...], 0.0)
    hw = jnp.dot(h, w2_ref[...], preferred_element_type=jnp.float32)
    y2_ref[...] = hw * dinv[:, None]


def _tc3_body(a_ref, y2_ref, dinv_ref, b2_ref, out_ref):
    dinv = dinv_ref[...]
    o = (a_ref[0] + a_ref[1] + y2_ref[...]) * dinv[:, None] + b2_ref[...]
    col = lax.broadcasted_iota(jnp.int32, o.shape, 1)
    neg = jnp.float32(-1e30)
    om = jnp.where(col < NCLS, o, neg)
    m = jnp.max(om, axis=1, keepdims=True)
    s = jnp.log(jnp.sum(jnp.exp(om - m), axis=1, keepdims=True))
    out_ref[...] = om - m - s


def _row_spec(width=None):
    if width is None:
        return pl.BlockSpec((_BLK,), lambda i: (i,))
    return pl.BlockSpec((_BLK, width), lambda i: (i, 0))


def _full_spec(shape):
    return pl.BlockSpec(shape, lambda i: tuple(0 for _ in shape))


_tc1 = pl.pallas_call(
    _tc1_body,
    grid=(_GRID,),
    in_specs=[
        pl.BlockSpec((NW, _BLK), lambda i: (0, i)),
        _row_spec(F_IN),
        _full_spec((F_IN, HID)),
    ],
    out_specs=[_row_spec(HID), _row_spec()],
    out_shape=[
        jax.ShapeDtypeStruct((N_PAD, HID), jnp.float32),
        jax.ShapeDtypeStruct((N_PAD,), jnp.float32),
    ],
)

_tc2 = pl.pallas_call(
    _tc2_body,
    grid=(_GRID,),
    in_specs=[
        pl.BlockSpec((NC, _BLK, HID), lambda i: (0, i, 0)),
        _row_spec(HID), _row_spec(),
        _full_spec((HID, HID)),
        _full_spec((1, HID)),
    ],
    out_specs=[_row_spec(HID)],
    out_shape=[jax.ShapeDtypeStruct((N_PAD, HID), jnp.float32)],
)

_tc3 = pl.pallas_call(
    _tc3_body,
    grid=(_GRID,),
    in_specs=[
        pl.BlockSpec((NC, _BLK, HID), lambda i: (0, i, 0)),
        _row_spec(HID), _row_spec(),
        _full_spec((1, HID)),
    ],
    out_specs=[_row_spec(HID)],
    out_shape=[jax.ShapeDtypeStruct((N_PAD, HID), jnp.float32)],
)


@jax.jit
def kernel(x, edge_index, W1, b1, W2, b2):
    src = edge_index[0]
    dst = edge_index[1]
    pad = E_PAD - E
    src_p = jnp.concatenate([src, jnp.zeros((pad,), jnp.int32)])
    dst_p = jnp.concatenate([dst, jnp.full((pad,), N, jnp.int32)])
    src3 = src_p.reshape(NW, NB, CH)
    dst3 = dst_p.reshape(NW, NB, CH)

    x_p = jnp.zeros((N_PAD, F_IN), jnp.float32).at[:N].set(x)
    w2_p = jnp.zeros((HID, HID), jnp.float32).at[:, :NCLS].set(W2)
    b1_p = jnp.reshape(b1, (1, HID))
    b2_p = jnp.zeros((1, HID), jnp.float32).at[0, :NCLS].set(b2)

    degp = _deg_kernel()(dst_p)
    y1, dinv = _tc1(degp, x_p, W1)

    acc1 = _scatter_kernel()(src3, dst3, y1)
    (y2,) = _tc2(acc1, y1, dinv, w2_p, b1_p)

    acc2 = _scatter_kernel()(src3, dst3, y2)
    (out,) = _tc3(acc2, y2, dinv, b2_p)

    return out[:N, :NCLS]


# speedup vs baseline: 1.6081x; 1.6081x over previous
import functools

import jax
import jax.numpy as jnp
from jax import lax
from jax.experimental import pallas as pl
from jax.experimental.pallas import tpu as pltpu
from jax.experimental.pallas import tpu_sc as plsc

N = 10000
E = 320000
F_IN = 128
HID = 64
NCLS = 40

NC = 2
NS = 16
NW = NC * NS
N_PAD = 10240
CH = 128
PER_W = 10112
E_PAD = PER_W * NW
NB = PER_W // CH
NCLS_P = 48
ROWS_PER_TILE = N_PAD // NS


def _wid():
    return lax.axis_index("s") * NC + lax.axis_index("c")


def _sc_degree(dst_hbm, out_hbm, idx_v, deg_v, sem):
    cid = lax.axis_index("c")
    sid = lax.axis_index("s")
    wid = sid * NC + cid

    def zero_body(i, _):
        deg_v[pl.ds(i * 16, 16)] = jnp.zeros((16,), jnp.float32)
        return 0
    lax.fori_loop(0, N_PAD // 16, zero_body, 0)

    pltpu.sync_copy(dst_hbm.at[pl.ds(wid * PER_W, PER_W)], idx_v)

    ones = jnp.ones((16,), jnp.float32)

    def count_body(i, _):
        idx = idx_v[pl.ds(i * 16, 16)]
        plsc.addupdate_scatter(deg_v, [idx], ones)
        return 0
    lax.fori_loop(0, PER_W // 16, count_body, 0)

    pltpu.sync_copy(deg_v, out_hbm.at[wid])


def _sc_scatter(D, src_hbm, dst_hbm, y_hbm, out_hbm, idx_s, idx_d,
                rows0, rows1, zbuf, acc_sh, sem0, sem1, semi):
    cid = lax.axis_index("c")
    sid = lax.axis_index("s")
    wid = sid * NC + cid

    cps = pltpu.async_copy(src_hbm.at[wid], idx_s, semi)
    cpd = pltpu.async_copy(dst_hbm.at[wid], idx_d, semi)

    def zero_zb(i, _):
        r = i // (D // 16)
        c = i % (D // 16)
        zbuf[r, pl.ds(c * 16, 16)] = jnp.zeros((16,), jnp.float32)
        return 0
    lax.fori_loop(0, CH * D // 16, zero_zb, 0)

    def zero_acc(i, _):
        pltpu.sync_copy(
            zbuf, acc_sh.at[pl.ds(sid * ROWS_PER_TILE + i * CH, CH), :])
        return 0
    lax.fori_loop(0, ROWS_PER_TILE // CH, zero_acc, 0)

    cps.wait()
    cpd.wait()
    plsc.subcore_barrier()

    def gather(j, buf, sem):
        pltpu.async_copy(y_hbm.at[idx_s.at[j]], buf, sem)

    def gwait(buf, sem):
        pltpu.make_async_copy(y_hbm.at[idx_s.at[0]], buf, sem).wait()

    def scat(j, buf):
        pltpu.sync_copy(buf, acc_sh.at[idx_d.at[j]], add=True)

    gather(0, rows0, sem0)

    def pair_body(jj, _):
        c0 = 2 * jj
        gather(c0 + 1, rows1, sem1)
        gwait(rows0, sem0)
        scat(c0, rows0)

        @pl.when(c0 + 2 < NB)
        def _():
            gather(c0 + 2, rows0, sem0)
        gwait(rows1, sem1)
        scat(c0 + 1, rows1)
        return 0
    lax.fori_loop(0, NB // 2, pair_body, 0)

    gwait(rows0, sem0)
    scat(NB - 1, rows0)

    plsc.subcore_barrier()
    pltpu.sync_copy(
        acc_sh.at[pl.ds(sid * ROWS_PER_TILE, ROWS_PER_TILE), :],
        out_hbm.at[cid, pl.ds(sid * ROWS_PER_TILE, ROWS_PER_TILE), :],
    )


@functools.cache
def _sc_mesh():
    return plsc.VectorSubcoreMesh(
        core_axis_name="c", subcore_axis_name="s",
        num_cores=NC, num_subcores=NS)


@functools.cache
def _deg_kernel():
    return pl.kernel(
        _sc_degree,
        out_type=jax.ShapeDtypeStruct((NW, N_PAD), jnp.float32),
        mesh=_sc_mesh(),
        compiler_params=pltpu.CompilerParams(
            needs_layout_passes=False, use_tc_tiling_on_sc=False),
        scratch_types=[
            pltpu.VMEM((PER_W,), jnp.int32),
            pltpu.VMEM((N_PAD,), jnp.float32),
            pltpu.SemaphoreType.DMA,
        ],
    )


@functools.cache
def _scatter_kernel(D):
    return pl.kernel(
        functools.partial(_sc_scatter, D),
        out_type=jax.ShapeDtypeStruct((NC, N_PAD, D), jnp.float32),
        mesh=_sc_mesh(),
        compiler_params=pltpu.CompilerParams(
            needs_layout_passes=False, use_tc_tiling_on_sc=False),
        scratch_types=(
            [pltpu.VMEM((NB, CH), jnp.int32)] * 2
            + [pltpu.VMEM((CH, D), jnp.float32)] * 3
            + [pltpu.VMEM_SHARED((N_PAD, D), jnp.float32)]
            + [pltpu.SemaphoreType.DMA] * 3
        ),
    )


_BLK = 2048
_GRID = N_PAD // _BLK


def _tc1_body(dp_ref, x_ref, w1_ref, y1_ref, dinv_ref):
    deg = jnp.sum(dp_ref[...], axis=0) + 1.0
    dinv = lax.rsqrt(deg)
    dinv_ref[...] = dinv
    xw = jnp.dot(x_ref[...], w1_ref[...], preferred_element_type=jnp.float32)
    y1_ref[...] = xw * dinv[:, None]


def _tc2_body(a_ref, y1_ref, dinv_ref, w2_ref, b1_ref, y2_ref):
    dinv = dinv_ref[...]
    pre = (a_ref[0] + a_ref[1] + y1_ref[...]) * dinv[:, None]
    h = jnp.maximum(pre + b1_ref[...], 0.0)
    hw = jnp.dot(h, w2_ref[...], preferred_element_type=jnp.float32)
    y2_ref[...] = hw * dinv[:, None]


def _tc3_body(a_ref, y2_ref, dinv_ref, b2_ref, out_ref):
    dinv = dinv_ref[...]
    o = (a_ref[0] + a_ref[1] + y2_ref[...]) * dinv[:, None] + b2_ref[...]
    col = lax.broadcasted_iota(jnp.int32, o.shape, 1)
    neg = jnp.float32(-1e30)
    om = jnp.where(col < NCLS, o, neg)
    m = jnp.max(om, axis=1, keepdims=True)
    s = jnp.log(jnp.sum(jnp.exp(om - m), axis=1, keepdims=True))
    out_ref[...] = om - m - s


def _row_spec(width=None):
    if width is None:
        return pl.BlockSpec((_BLK,), lambda i: (i,))
    return pl.BlockSpec((_BLK, width), lambda i: (i, 0))


def _full_spec(shape):
    return pl.BlockSpec(shape, lambda i: tuple(0 for _ in shape))


_tc1 = pl.pallas_call(
    _tc1_body,
    grid=(_GRID,),
    in_specs=[
        pl.BlockSpec((NW, _BLK), lambda i: (0, i)),
        _row_spec(F_IN),
        _full_spec((F_IN, HID)),
    ],
    out_specs=[_row_spec(HID), _row_spec()],
    out_shape=[
        jax.ShapeDtypeStruct((N_PAD, HID), jnp.float32),
        jax.ShapeDtypeStruct((N_PAD,), jnp.float32),
    ],
)

_tc2 = pl.pallas_call(
    _tc2_body,
    grid=(_GRID,),
    in_specs=[
        pl.BlockSpec((NC, _BLK, HID), lambda i: (0, i, 0)),
        _row_spec(HID), _row_spec(),
        _full_spec((HID, NCLS_P)),
        _full_spec((1, HID)),
    ],
    out_specs=[_row_spec(NCLS_P)],
    out_shape=[jax.ShapeDtypeStruct((N_PAD, NCLS_P), jnp.float32)],
)

_tc3 = pl.pallas_call(
    _tc3_body,
    grid=(_GRID,),
    in_specs=[
        pl.BlockSpec((NC, _BLK, NCLS_P), lambda i: (0, i, 0)),
        _row_spec(NCLS_P), _row_spec(),
        _full_spec((1, NCLS_P)),
    ],
    out_specs=[_row_spec(NCLS_P)],
    out_shape=[jax.ShapeDtypeStruct((N_PAD, NCLS_P), jnp.float32)],
)


@jax.jit
def kernel(x, edge_index, W1, b1, W2, b2):
    src = edge_index[0]
    dst = edge_index[1]
    pad = E_PAD - E
    src_p = jnp.concatenate([src, jnp.zeros((pad,), jnp.int32)])
    dst_p = jnp.concatenate([dst, jnp.full((pad,), N, jnp.int32)])
    src3 = src_p.reshape(NW, NB, CH)
    dst3 = dst_p.reshape(NW, NB, CH)

    x_p = jnp.zeros((N_PAD, F_IN), jnp.float32).at[:N].set(x)
    w2_p = jnp.zeros((HID, NCLS_P), jnp.float32).at[:, :NCLS].set(W2)
    b1_p = jnp.reshape(b1, (1, HID))
    b2_p = jnp.zeros((1, NCLS_P), jnp.float32).at[0, :NCLS].set(b2)

    degp = _deg_kernel()(dst_p)
    y1, dinv = _tc1(degp, x_p, W1)

    acc1 = _scatter_kernel(HID)(src3, dst3, y1)
    (y2,) = _tc2(acc1, y1, dinv, w2_p, b1_p)

    acc2 = _scatter_kernel(NCLS_P)(src3, dst3, y2)
    (out,) = _tc3(acc2, y2, dinv, b2_p)

    return out[:N, :NCLS]
